# baseline (speedup 1.0000x reference)
import functools

import jax
import jax.numpy as jnp
from jax import lax
from jax.experimental import pallas as pl
from jax.experimental.pallas import tpu as pltpu
from jax.experimental.pallas import tpu_sc as plsc

NC = 2
NS = 16
LANES = 16
B = 128


def _sc_message_pass(N, D, E):
    nblk = E // B
    assert nblk * B == E
    nworkers = NC * NS
    nfull = nblk // nworkers
    nextra = nblk % nworkers
    ch = 80
    nch = N // ch
    assert nch * ch == N
    rounds = -(-nch // NS)
    assert nfull % 2 == 0

    mesh = plsc.VectorSubcoreMesh(core_axis_name="c", subcore_axis_name="s")

    @functools.partial(
        pl.kernel,
        mesh=mesh,
        out_type=jax.ShapeDtypeStruct((NC, N, D), jnp.float32),
        scratch_types=[
            pltpu.VMEM((2, B), jnp.int32),
            pltpu.VMEM((2, B), jnp.int32),
            pltpu.VMEM((B, D), jnp.float32),
            pltpu.VMEM((B, D), jnp.float32),
            pltpu.VMEM((B, D), jnp.float32),
            pltpu.VMEM_SHARED((N, D), jnp.float32),
            pltpu.SemaphoreType.DMA,
            pltpu.SemaphoreType.DMA,
            pltpu.SemaphoreType.DMA,
            pltpu.SemaphoreType.DMA,
            pltpu.SemaphoreType.DMA,
            pltpu.SemaphoreType.DMA,
            pltpu.SemaphoreType.DMA,
            pltpu.SemaphoreType.DMA,
        ],
    )
    def k(node_hbm, eidx_hbm, edge_hbm, part_hbm,
          ix0, ix1, m0, m1, g, acc,
          is0, is1, es0, es1, gsem, ssem, ws0, ws1):
        wsem = (ws0, ws1)
        idx = (ix0, ix1)
        m = (m0, m1)
        isem, esem = (is0, is1), (es0, es1)
        cid = lax.axis_index("c")
        sid = lax.axis_index("s")
        wid = sid * NC + cid

        def zrow(r, _):
            for c in range(D // LANES):
                m0[r, pl.ds(c * LANES, LANES)] = jnp.zeros((LANES,), jnp.float32)
            return 0
        lax.fori_loop(0, B, zrow, 0)
        for kk in range(rounds):
            j = kk * NS + sid
            @pl.when(j < nch)
            def _():
                pltpu.sync_copy(m0.at[pl.ds(0, ch)], acc.at[pl.ds(j * ch, ch)])
        plsc.subcore_barrier()

        def issue(blk, b):
            pltpu.async_copy(eidx_hbm.at[blk], idx[b], isem[b])
            pltpu.async_copy(edge_hbm.at[pl.ds(blk * B, B)], m[b], esem[b])

        def wait_idx(b):
            pltpu.make_async_copy(eidx_hbm.at[0], idx[b], isem[b]).wait()

        def wait_edge(b):
            pltpu.make_async_copy(edge_hbm.at[pl.ds(0, B)], m[b], esem[b]).wait()

        def compute(b, h):
            mb = m[b]
            base = h * (B // 2)

            def row(r, _):
                for rr in range(8):
                    for c in range(D // LANES):
                        sl = pl.ds(c * LANES, LANES)
                        r2 = base + r * 8 + rr
                        mb[r2, sl] = jnp.maximum(mb[r2, sl] + g[r2, sl], 0.0)
                return 0
            lax.fori_loop(0, B // 16, row, 0)

        def gather_half(b, h, sem):
            return pltpu.async_copy(
                node_hbm.at[idx[b].at[0, pl.ds(h * (B // 2), B // 2)]],
                g.at[pl.ds(h * (B // 2), B // 2)], sem)

        def scatter(b):
            pltpu.sync_copy(m[b], acc.at[idx[b].at[1]], add=True)

        first = wid * nfull
        issue(first, 0)

        def body(i2, _):
            for b in (0, 1):
                i = i2 * 2 + b
                blk = first + i
                q = 1 - b
                wait_idx(b)
                gda = gather_half(b, 0, gsem)
                gdb = gather_half(b, 1, ssem)
                @pl.when(i >= 1)
                def _():
                    scatter(q)
                @pl.when(i + 1 < nfull)
                def _():
                    issue(blk + 1, q)
                wait_edge(b)
                gda.wait()
                compute(b, 0)
                gdb.wait()
                compute(b, 1)
            return 0
        lax.fori_loop(0, nfull // 2, body, 0)
        scatter(1)

        if nextra:
            @pl.when(wid < nextra)
            def _():
                blk = nworkers * nfull + wid
                issue(blk, 0)
                wait_idx(0)
                wait_edge(0)
                gda = gather_half(0, 0, gsem)
                gdb = gather_half(0, 1, ssem)
                gda.wait()
                compute(0, 0)
                gdb.wait()
                compute(0, 1)
                scatter(0)

        plsc.subcore_barrier()
        wd = [None, None]

        def wchunk(kk, b):
            r0 = (kk * NS + sid) * ch
            if wd[b] is not None:
                wd[b].wait()
            pltpu.sync_copy(acc.at[pl.ds(r0, ch)], m[b].at[pl.ds(0, ch)])
            wd[b] = pltpu.async_copy(m[b].at[pl.ds(0, ch)],
                                     part_hbm.at[cid, pl.ds(r0, ch)], wsem[b])

        for kk in range(rounds):
            b = kk % 2
            if (kk + 1) * NS <= nch:
                wchunk(kk, b)
            else:
                @pl.when(kk * NS + sid < nch)
                def _():
                    wchunk(kk, b)
        for b in (0, 1):
            pltpu.make_async_copy(m[b].at[pl.ds(0, ch)],
                                  part_hbm.at[cid, pl.ds(0, ch)],
                                  wsem[b]).wait()

    return k


def _combine(x_ref, p_ref, o_ref):
    o_ref[...] = x_ref[...] + p_ref[0] + p_ref[1]


def kernel(node_feat, edge_index, edge_feat):
    N, D = node_feat.shape
    E = edge_feat.shape[0]
    nblk = E // B
    eidx = jnp.transpose(edge_index.reshape(2, nblk, B), (1, 0, 2))
    parts = _sc_message_pass(N, D, E)(node_feat, eidx, edge_feat)

    rb = 1000 if N % 1000 == 0 else N
    out = pl.pallas_call(
        _combine,
        grid=(N // rb,),
        in_specs=[
            pl.BlockSpec((rb, D), lambda i: (i, 0)),
            pl.BlockSpec((NC, rb, D), lambda i: (0, i, 0)),
        ],
        out_specs=pl.BlockSpec((rb, D), lambda i: (i, 0)),
        out_shape=jax.ShapeDtypeStruct((N, D), jnp.float32),
    )(node_feat, parts)
    return out


# speedup vs baseline: 1.0201x; 1.0201x over previous
import functools

import jax
import jax.numpy as jnp
from jax import lax
from jax.experimental import pallas as pl
from jax.experimental.pallas import tpu as pltpu
from jax.experimental.pallas import tpu_sc as plsc

NC = 2
NS = 16
LANES = 16
B = 128


def _sc_message_pass(N, D, E):
    nblk = E // B
    assert nblk * B == E
    nworkers = NC * NS
    nfull = nblk // nworkers
    nextra = nblk % nworkers
    ch = 80
    nch = N // ch
    assert nch * ch == N
    rounds = -(-nch // NS)
    assert nfull % 2 == 0

    mesh = plsc.VectorSubcoreMesh(core_axis_name="c", subcore_axis_name="s")

    @functools.partial(
        pl.kernel,
        mesh=mesh,
        out_type=jax.ShapeDtypeStruct((NC, N, D), jnp.float32),
        scratch_types=[
            pltpu.VMEM((2, B), jnp.int32),
            pltpu.VMEM((2, B), jnp.int32),
            pltpu.VMEM((B, D), jnp.float32),
            pltpu.VMEM((B, D), jnp.float32),
            pltpu.VMEM((B, D), jnp.float32),
            pltpu.VMEM_SHARED((N, D), jnp.float32),
            pltpu.SemaphoreType.DMA,
            pltpu.SemaphoreType.DMA,
            pltpu.SemaphoreType.DMA,
            pltpu.SemaphoreType.DMA,
            pltpu.SemaphoreType.DMA,
            pltpu.SemaphoreType.DMA,
        ],
    )
    def k(node_hbm, eidx_hbm, edge_hbm, part_hbm,
          ix0, ix1, m0, m1, g, acc,
          is0, is1, es0, es1, gsem, ssem):
        idx = (ix0, ix1)
        m = (m0, m1)
        isem, esem = (is0, is1), (es0, es1)
        cid = lax.axis_index("c")
        sid = lax.axis_index("s")
        wid = sid * NC + cid

        def zrow(r, _):
            for c in range(D // LANES):
                m0[r, pl.ds(c * LANES, LANES)] = jnp.zeros((LANES,), jnp.float32)
            return 0
        lax.fori_loop(0, B, zrow, 0)
        for kk in range(rounds):
            j = kk * NS + sid
            @pl.when(j < nch)
            def _():
                pltpu.sync_copy(m0.at[pl.ds(0, ch)], acc.at[pl.ds(j * ch, ch)])
        plsc.subcore_barrier()

        def issue(blk, b):
            pltpu.async_copy(eidx_hbm.at[blk], idx[b], isem[b])
            pltpu.async_copy(edge_hbm.at[pl.ds(blk * B, B)], m[b], esem[b])

        def wait_idx(b):
            pltpu.make_async_copy(eidx_hbm.at[0], idx[b], isem[b]).wait()

        def wait_edge(b):
            pltpu.make_async_copy(edge_hbm.at[pl.ds(0, B)], m[b], esem[b]).wait()

        def compute(b, h):
            mb = m[b]
            base = h * (B // 2)

            def row(r, _):
                for rr in range(4):
                    for c in range(D // LANES):
                        sl = pl.ds(c * LANES, LANES)
                        r2 = base + r * 4 + rr
                        mb[r2, sl] = jnp.maximum(mb[r2, sl] + g[r2, sl], 0.0)
                return 0
            lax.fori_loop(0, B // 8, row, 0)

        def gather_half(b, h, sem):
            return pltpu.async_copy(
                node_hbm.at[idx[b].at[0, pl.ds(h * (B // 2), B // 2)]],
                g.at[pl.ds(h * (B // 2), B // 2)], sem)

        def scatter(b):
            pltpu.sync_copy(m[b], acc.at[idx[b].at[1]], add=True)

        first = wid * nfull
        issue(first, 0)

        def body(i2, _):
            for b in (0, 1):
                i = i2 * 2 + b
                blk = first + i
                q = 1 - b
                wait_idx(b)
                gda = gather_half(b, 0, gsem)
                gdb = gather_half(b, 1, ssem)
                @pl.when(i >= 1)
                def _():
                    scatter(q)
                @pl.when(i + 1 < nfull)
                def _():
                    issue(blk + 1, q)
                wait_edge(b)
                gda.wait()
                compute(b, 0)
                gdb.wait()
                compute(b, 1)
            return 0
        lax.fori_loop(0, nfull // 2, body, 0)
        scatter(1)

        if nextra:
            @pl.when(wid < nextra)
            def _():
                blk = nworkers * nfull + wid
                issue(blk, 0)
                wait_idx(0)
                wait_edge(0)
                gda = gather_half(0, 0, gsem)
                gdb = gather_half(0, 1, ssem)
                gda.wait()
                compute(0, 0)
                gdb.wait()
                compute(0, 1)
                scatter(0)

        plsc.subcore_barrier()
        for kk in range(rounds):
            j = kk * NS + sid
            @pl.when(j < nch)
            def _():
                r0 = j * ch
                pltpu.sync_copy(acc.at[pl.ds(r0, ch)], m0.at[pl.ds(0, ch)])
                pltpu.sync_copy(m0.at[pl.ds(0, ch)],
                                part_hbm.at[cid, pl.ds(r0, ch)])

    return k


def _combine(x_ref, p_ref, o_ref):
    o_ref[...] = x_ref[...] + p_ref[0] + p_ref[1]


def kernel(node_feat, edge_index, edge_feat):
    N, D = node_feat.shape
    E = edge_feat.shape[0]
    nblk = E // B
    eidx = jnp.transpose(edge_index.reshape(2, nblk, B), (1, 0, 2))
    parts = _sc_message_pass(N, D, E)(node_feat, eidx, edge_feat)

    rb = 1000 if N % 1000 == 0 else N
    out = pl.pallas_call(
        _combine,
        grid=(N // rb,),
        in_specs=[
            pl.BlockSpec((rb, D), lambda i: (i, 0)),
            pl.BlockSpec((NC, rb, D), lambda i: (0, i, 0)),
        ],
        out_specs=pl.BlockSpec((rb, D), lambda i: (i, 0)),
        out_shape=jax.ShapeDtypeStruct((N, D), jnp.float32),
    )(node_feat, parts)
    return out
